# Initial kernel scaffold; baseline (speedup 1.0000x reference)
import jax
import jax.numpy as jnp
from jax.experimental import pallas as pl


def kernel(x, edge_index, W1, b1, W2, b2):
    raise NotImplementedError("write your pallas kernel here")


# speedup vs baseline: 1.2663x; 1.2663x over previous
import jax
import jax.numpy as jnp
from jax import lax
from jax.experimental import pallas as pl
from jax.experimental.pallas import tpu as pltpu
from jax.experimental.pallas import tpu_sc as plsc

_NC = 2
_NS = 16
_BLK = 128
_C = 2432
_NACC = 2560
_NCH = 5
_NBT = 21
_P = _NBT * _NC * _NS * _BLK
_TB = _P // _BLK
_DEGW = 16


def _make_sc_agg(d):
    rpt = _NACC // _NS

    def body(h_hbm, row_hbm, col_hbm, zeros_hbm, o_hbm,
             rbuf, cbuf, rows_v, stage, sem, acc):
        c = lax.axis_index("c")
        s = lax.axis_index("s")
        g = c * _NS + s
        for q in range(_NCH):
            pltpu.sync_copy(zeros_hbm, stage)
            pltpu.sync_copy(stage, acc.at[pl.ds(s * rpt, rpt)])
            plsc.subcore_barrier()
            for i in range(_NBT):
                off = q * _P + (g * _NBT + i) * _BLK
                pltpu.sync_copy(row_hbm.at[pl.ds(off, _BLK)], rbuf)
                pltpu.sync_copy(col_hbm.at[pl.ds(off, _BLK)], cbuf)
                pltpu.async_copy(h_hbm.at[rbuf], rows_v, sem).wait()
                pltpu.sync_copy(rows_v, acc.at[cbuf], add=True)
            plsc.subcore_barrier()
            pltpu.sync_copy(acc.at[pl.ds(s * rpt, rpt)], stage)
            pltpu.sync_copy(stage, o_hbm.at[c, q, pl.ds(s * rpt, rpt)])
            plsc.subcore_barrier()

    mesh = plsc.VectorSubcoreMesh(core_axis_name="c", subcore_axis_name="s")
    return pl.kernel(
        body,
        out_type=jax.ShapeDtypeStruct((_NC, _NCH, _NACC, d), jnp.float32),
        mesh=mesh,
        compiler_params=pltpu.CompilerParams(use_tc_tiling_on_sc=(d % 128 == 0)),
        scratch_types=[
            pltpu.VMEM((_BLK,), jnp.int32),
            pltpu.VMEM((_BLK,), jnp.int32),
            pltpu.VMEM((_BLK, d), jnp.float32),
            pltpu.VMEM((rpt, d), jnp.float32),
            pltpu.SemaphoreType.DMA,
            pltpu.VMEM_SHARED((_NACC, d), jnp.float32),
        ],
    )


def _sc_aggregate(table, rows, cols, d):
    f = _make_sc_agg(d)
    zeros = jnp.zeros((_NACC // _NS, d), jnp.float32)
    return f(table, rows, cols, zeros)


def _tc1_body(x_ref, w1_ref, degp_ref, h1p_ref, dis_ref):
    x = x_ref[...]
    xm = x - jnp.mean(x, axis=0, keepdims=True)
    rn = jnp.sqrt(1e-6 + jnp.mean(jnp.sum(xm * xm, axis=1)))
    hn = xm / rn
    deg = degp_ref[0, :, 0:1] + degp_ref[1, :, 0:1] + 1.0
    dis = lax.rsqrt(deg)
    h1 = jnp.dot(hn, w1_ref[...], preferred_element_type=jnp.float32)
    h1p_ref[...] = h1 * dis
    dis_ref[...] = dis


def _tc2_body(agg_ref, h1p_ref, dis_ref, b1_ref, w2_ref, h2p_ref):
    dis = dis_ref[...]
    z = (agg_ref[0] + agg_ref[1] + h1p_ref[...]) * dis + b1_ref[...]
    a = jnp.maximum(z, 0.0)
    h2p_ref[...] = jnp.dot(a, w2_ref[...], preferred_element_type=jnp.float32) * dis


def _tc3_body(agg_ref, h2p_ref, dis_ref, b2_ref, out_ref):
    out_ref[...] = ((agg_ref[0] + agg_ref[1] + h2p_ref[...])
                    * dis_ref[...] + b2_ref[...])


def _slots(n, edge_index):
    row = edge_index[0]
    col = edge_index[1]
    zrow = jnp.int32(n)
    selfm = row == col
    colp = jnp.where(selfm, jnp.int32(n), col)
    rowp = jnp.where(selfm, zrow, row)
    order = jnp.argsort(colp)
    scol = jnp.concatenate([colp[order], jnp.full((_P,), jnp.int32(2**30))])
    srow = jnp.concatenate([rowp[order], jnp.full((_P,), zrow)])
    starts = jnp.searchsorted(scol[: col.shape[0]],
                              jnp.arange(_NCH, dtype=jnp.int32) * _C)
    s_idx = (jnp.arange(_P, dtype=jnp.int32) % _BLK)
    col_parts, row_parts, deg_parts = [], [], []
    for q in range(_NCH):
        wc = lax.dynamic_slice(scol, (starts[q],), (_P,))
        wr = lax.dynamic_slice(srow, (starts[q],), (_P,))
        wc = wc.reshape(_BLK, _TB).T.reshape(-1)
        wr = wr.reshape(_BLK, _TB).T.reshape(-1)
        lo = q * _C
        hi = min((q + 1) * _C, n)
        ok = (wc >= lo) & (wc < hi)
        col_parts.append(jnp.where(ok, wc - lo, _C + s_idx))
        row_parts.append(jnp.where(ok, wr, zrow))
        deg_parts.append(jnp.where(ok, s_idx, _BLK + s_idx))
    return (jnp.concatenate(col_parts), jnp.concatenate(row_parts),
            jnp.concatenate(deg_parts))


def _chunks_to_nodes(part, n):
    return jnp.concatenate([part[:, q, :_C] for q in range(_NCH)], axis=1)[:, :n]


def kernel(x, edge_index, W1, b1, W2, b2):
    n, d_in = x.shape
    d_h = W1.shape[1]
    d_out = W2.shape[1]

    cols, rows, degsrc = _slots(n, edge_index)

    deg_table = jnp.concatenate([jnp.ones((_BLK, _DEGW), jnp.float32),
                                 jnp.zeros((_BLK, _DEGW), jnp.float32)])
    degp = _sc_aggregate(deg_table, degsrc, cols, _DEGW)
    degcat = _chunks_to_nodes(degp, n)

    h1p, dis = pl.pallas_call(
        _tc1_body,
        out_shape=[
            jax.ShapeDtypeStruct((n, d_h), jnp.float32),
            jax.ShapeDtypeStruct((n, 1), jnp.float32),
        ],
    )(x, W1, degcat)

    h1p_pad = jnp.concatenate([h1p, jnp.zeros((8, d_h), jnp.float32)])
    agg1 = _sc_aggregate(h1p_pad, rows, cols, d_h)

    h2p = pl.pallas_call(
        _tc2_body,
        out_shape=jax.ShapeDtypeStruct((n, d_out), jnp.float32),
    )(_chunks_to_nodes(agg1, n), h1p, dis, b1.reshape(1, d_h), W2)

    h2p_pad = jnp.concatenate([h2p, jnp.zeros((8, d_out), jnp.float32)])
    agg2 = _sc_aggregate(h2p_pad, rows, cols, d_out)

    out = pl.pallas_call(
        _tc3_body,
        out_shape=jax.ShapeDtypeStruct((n, d_out), jnp.float32),
    )(_chunks_to_nodes(agg2, n), h2p, dis, b2.reshape(1, d_out))

    return out
